# baseline (speedup 1.0000x reference)
import functools
import jax
import jax.numpy as jnp
from jax import lax
from jax.experimental import pallas as pl
from jax.experimental.pallas import tpu as pltpu
from jax.experimental.pallas import tpu_sc as plsc

NN = 10000
EE = 160000
H = 128
NC = 2
NS = 16
NW = NC * NS
CB = 128
NCHUNK = EE // CB
KMAX = -(-NCHUNK // NW)
CBR = 64
NCHUNK_R = EE // CBR
KMAX_R = -(-NCHUNK_R // NW)
KMAX1 = -(-NCHUNK // NS)
RPT = 632
NP = RPT * NS

_mesh = plsc.VectorSubcoreMesh(
    core_axis_name="c", subcore_axis_name="s", num_cores=NC, num_subcores=NS)

_f32 = jnp.float32


def _wid():
    return lax.axis_index("s") * NC + lax.axis_index("c")


def _zero_fill(buf, rows):
    z = jnp.zeros((16,), _f32)

    def row(i, _):
        for cc in range(buf.shape[1] // 16):
            buf[i, pl.ds(cc * 16, 16)] = z
        return 0

    lax.fori_loop(0, rows, row, 0)


def _stripe_init(buf, acc):
    s = lax.axis_index("s")
    nb = buf.shape[0]
    _zero_fill(buf, nb)
    base = s * RPT
    for j in range(RPT // nb):
        pltpu.sync_copy(buf, acc.at[pl.ds(base + j * nb, nb)])
    rem = RPT % nb
    if rem:
        pltpu.sync_copy(buf.at[pl.ds(0, rem)],
                        acc.at[pl.ds(base + RPT - rem, rem)])


def _stripe_dump(acc, out):
    c = lax.axis_index("c")
    s = lax.axis_index("s")
    base = s * RPT
    for j in range(RPT // CB):
        pltpu.sync_copy(acc.at[pl.ds(base + j * CB, CB)],
                        out.at[c, pl.ds(base + j * CB, CB)])
    rem = RPT % CB
    if rem:
        pltpu.sync_copy(acc.at[pl.ds(base + RPT - rem, rem)],
                        out.at[c, pl.ds(base + RPT - rem, rem)])


@functools.partial(
    pl.kernel,
    out_type=(jax.ShapeDtypeStruct((EE,), _f32),
              jax.ShapeDtypeStruct((NC * NP, H), _f32)),
    mesh=_mesh,
    scratch_types=[
        pltpu.VMEM((CB,), jnp.int32),
        pltpu.VMEM((CB,), jnp.int32),
        pltpu.VMEM((CB,), jnp.int32),
        pltpu.VMEM((CB, H), _f32),
        pltpu.VMEM((CB, H), _f32),
        pltpu.VMEM((CB, H), _f32),
        pltpu.VMEM((CB,), _f32),
        pltpu.VMEM_SHARED((NP, H), _f32),
        pltpu.SemaphoreType.DMA,
        pltpu.SemaphoreType.DMA,
    ],
)
def _sc_degree(src_h, dst_h, inv_h, cnt_h, idx_s, idx_d, idx2, ones_b, buf_a,
               buf_b, buf_o, cnt, sem_a, sem_b):
    c = lax.axis_index("c")
    s = lax.axis_index("s")
    wid = _wid()
    one = jnp.ones((16,), _f32)

    def fill(i, _):
        for cc in range(H // 16):
            ones_b[i, pl.ds(cc * 16, 16)] = one
        return 0

    lax.fori_loop(0, CB, fill, 0)
    _stripe_init(buf_a, cnt)
    plsc.subcore_barrier()

    def count_step(k, _):
        cid = k * NS + s

        @pl.when(cid < NCHUNK)
        def _():
            base = pl.multiple_of(cid * CB, CB)
            pltpu.sync_copy(src_h.at[pl.ds(base, CB)], idx_s)
            pltpu.sync_copy(dst_h.at[pl.ds(base, CB)], idx_d)
            pltpu.sync_copy(ones_b, cnt.at[idx_s], add=True)
            pltpu.sync_copy(ones_b, cnt.at[idx_d], add=True)

        return 0

    lax.fori_loop(0, KMAX1, count_step, 0)
    plsc.subcore_barrier()

    for j in range(RPT // CB):
        pltpu.sync_copy(cnt.at[pl.ds(s * RPT + j * CB, CB)],
                        cnt_h.at[pl.ds(c * NP + s * RPT + j * CB, CB)])
    rem = RPT % CB
    pltpu.sync_copy(cnt.at[pl.ds(s * RPT + RPT - rem, rem)],
                    cnt_h.at[pl.ds(c * NP + s * RPT + RPT - rem, rem)])
    plsc.subcore_barrier()

    lane = lax.iota(jnp.int32, 16)
    off = jnp.full((16,), NP, jnp.int32) * c

    def inv_step(k, _):
        cid = k * NW + wid

        @pl.when(cid < NCHUNK)
        def _():
            base = pl.multiple_of(cid * CB, CB)
            pltpu.sync_copy(src_h.at[pl.ds(base, CB)], idx_s)
            pltpu.sync_copy(dst_h.at[pl.ds(base, CB)], idx_d)

            def addoff(src_ref):
                def go(g, _):
                    sl = pl.ds(g * 16, 16)
                    idx2[sl] = src_ref[sl] + off
                    return 0

                lax.fori_loop(0, CB // 16, go, 0)

            addoff(idx_s)
            cpa = pltpu.async_copy(cnt_h.at[idx2], buf_a, sem_a)
            cpa.wait()
            addoff(idx_d)
            cpb = pltpu.async_copy(cnt_h.at[idx2], buf_b, sem_b)
            cpb.wait()

            def grp(g, _):
                def rowf(r, acc):
                    j = g * 16 + r
                    a = buf_a[j, pl.ds(0, 16)]
                    b = buf_b[j, pl.ds(0, 16)]
                    iv = 1.0 / jnp.maximum(a + b - 2.0, 1.0)
                    return jnp.where(lane == r, iv, acc)

                buf_o[pl.ds(g * 16, 16)] = lax.fori_loop(
                    0, 16, rowf, jnp.zeros((16,), _f32))
                return 0

            lax.fori_loop(0, CB // 16, grp, 0)
            pltpu.sync_copy(buf_o, inv_h.at[pl.ds(base, CB)])

        return 0

    lax.fori_loop(0, KMAX, inv_step, 0)


EPT = EE // NW
CP = 48
NFULL = EPT // CP
TAIL = EPT - NFULL * CP
NPAIR = NFULL // 2


def _make_edge_up(has_r):
    nbig = 3 if has_r else 2
    scratch = (
        [pltpu.VMEM((CP,), jnp.int32)] * 4
        + [pltpu.VMEM((TAIL,), jnp.int32)] * 2
        + [pltpu.VMEM((CP, H), _f32)] * (2 * nbig)
        + [pltpu.VMEM_SHARED((NP, H), _f32),
           pltpu.SemaphoreType.DMA,
           pltpu.SemaphoreType.DMA]
    )

    def body(p_h, q_h, *rest):
        if has_r:
            (r_h, src_h, dst_h, he_h, s_h, ia0, id0, ia1, id1, iat, idt,
             a0, b0, c0, a1, b1, c1, acc, sem0, sem1) = rest
        else:
            (src_h, dst_h, he_h, s_h, ia0, id0, ia1, id1, iat, idt,
             a0, b0, a1, b1, acc, sem0, sem1) = rest
            c0 = c1 = None
        wid = _wid()
        tb = wid * EPT
        _stripe_init(a0, acc)
        plsc.subcore_barrier()
        sets = [(ia0, id0, a0, b0, c0, sem0), (ia1, id1, a1, b1, c1, sem1)]

        def issue(step, S):
            ia, idd, a, b, c, sem = sets[S]
            base = tb + step * CP
            pltpu.sync_copy(src_h.at[pl.ds(base, CP)], ia)
            pltpu.sync_copy(dst_h.at[pl.ds(base, CP)], idd)
            pltpu.async_copy(p_h.at[ia], a, sem)
            pltpu.async_copy(q_h.at[idd], b, sem)
            if has_r:
                pltpu.sync_copy(r_h.at[pl.ds(base, CP)], c)

        def waits(S):
            _, _, a, b, _, sem = sets[S]
            pltpu.make_async_copy(p_h.at[pl.ds(0, CP)], a, sem).wait()
            pltpu.make_async_copy(p_h.at[pl.ds(0, CP)], b, sem).wait()

        def compute(S, nrows):
            _, _, a, b, c, _ = sets[S]

            def row(i, _):
                for cc in range(H // 16):
                    sl = pl.ds(cc * 16, 16)
                    v = a[i, sl] + b[i, sl]
                    if has_r:
                        v = v + c[i, sl]
                    a[i, sl] = jnp.maximum(v, 0.0)
                return 0

            lax.fori_loop(0, nrows, row, 0)

        def outputs(step, S):
            ia, idd, a, _, _, _ = sets[S]
            base = tb + step * CP
            pltpu.sync_copy(a, he_h.at[pl.ds(base, CP)])
            pltpu.sync_copy(a, acc.at[ia], add=True)
            pltpu.sync_copy(a, acc.at[idd], add=True)

        issue(0, 0)

        def pair(j, _):
            issue(2 * j + 1, 1)
            waits(0)
            compute(0, CP)
            outputs(2 * j, 0)

            @pl.when(2 * j + 2 < NFULL)
            def _():
                issue(2 * j + 2, 0)

            waits(1)
            compute(1, CP)
            outputs(2 * j + 1, 1)
            return 0

        lax.fori_loop(0, NPAIR, pair, 0)

        base = tb + NFULL * CP
        pltpu.sync_copy(src_h.at[pl.ds(base, TAIL)], iat)
        pltpu.sync_copy(dst_h.at[pl.ds(base, TAIL)], idt)
        pltpu.async_copy(p_h.at[iat], a0.at[pl.ds(0, TAIL)], sem0)
        pltpu.async_copy(q_h.at[idt], b0.at[pl.ds(0, TAIL)], sem0)
        if has_r:
            pltpu.sync_copy(r_h.at[pl.ds(base, TAIL)], c0.at[pl.ds(0, TAIL)])
        pltpu.make_async_copy(p_h.at[pl.ds(0, TAIL)],
                              a0.at[pl.ds(0, TAIL)], sem0).wait()
        pltpu.make_async_copy(p_h.at[pl.ds(0, TAIL)],
                              b0.at[pl.ds(0, TAIL)], sem0).wait()
        compute(0, TAIL)
        pltpu.sync_copy(a0.at[pl.ds(0, TAIL)], he_h.at[pl.ds(base, TAIL)])
        pltpu.sync_copy(a0.at[pl.ds(0, TAIL)], acc.at[iat], add=True)
        pltpu.sync_copy(a0.at[pl.ds(0, TAIL)], acc.at[idt], add=True)

        plsc.subcore_barrier()
        _stripe_dump(acc, s_h)

    return functools.partial(
        pl.kernel,
        out_type=(jax.ShapeDtypeStruct((EE, H), _f32),
                  jax.ShapeDtypeStruct((NC, NP, H), _f32)),
        mesh=_mesh,
        scratch_types=scratch,
    )(body)


_sc_edge_up0 = _make_edge_up(False)
_sc_edge_up1 = _make_edge_up(True)


def _make_edge_ref(write_rp):
    outs = [jax.ShapeDtypeStruct((NC, NP, H), _f32)]
    if write_rp:
        outs = [jax.ShapeDtypeStruct((EE, H), _f32)] + outs

    def body(t_h, u2_h, inv_h, bee_h, src_h, dst_h, *rest):
        if write_rp:
            (rp_h, agg_h, ia0, id0, ia1, id1, iat, idt,
             ts0, td0, v0, ts1, td1, v1, iv0, iv1,
             acc, sem0, sem1) = rest
        else:
            (agg_h, ia0, id0, ia1, id1, iat, idt,
             ts0, td0, v0, ts1, td1, v1, iv0, iv1,
             acc, sem0, sem1) = rest
        wid = _wid()
        tb = wid * EPT
        pltpu.sync_copy(bee_h, td0.at[0])
        bee_r = [td0[0, pl.ds(cc * 16, 16)] for cc in range(H // 16)]
        _stripe_init(ts0, acc)
        plsc.subcore_barrier()
        zi16 = jnp.zeros((16,), jnp.int32)
        sets = [(ia0, id0, ts0, td0, v0, iv0, sem0),
                (ia1, id1, ts1, td1, v1, iv1, sem1)]

        def issue(step, S):
            ia, idd, ts, td, v, iv, sem = sets[S]
            base = tb + step * CP
            pltpu.sync_copy(src_h.at[pl.ds(base, CP)], ia)
            pltpu.sync_copy(dst_h.at[pl.ds(base, CP)], idd)
            pltpu.async_copy(t_h.at[ia], ts, sem)
            pltpu.async_copy(t_h.at[idd], td, sem)
            pltpu.sync_copy(u2_h.at[pl.ds(base, CP)], v)
            pltpu.sync_copy(inv_h.at[pl.ds(base, CP)], iv)

        def waits(S):
            _, _, ts, td, _, _, sem = sets[S]
            pltpu.make_async_copy(t_h.at[pl.ds(0, CP)], ts, sem).wait()
            pltpu.make_async_copy(t_h.at[pl.ds(0, CP)], td, sem).wait()

        def compute(S, ngrp):
            _, _, ts, td, v, iv_b, _ = sets[S]

            def grp(g, _):
                iv16 = iv_b[pl.ds(g * 16, 16)]

                def rowf(r, _):
                    i = g * 16 + r
                    iv = lax.gather(
                        iv16, (zi16 + r)[:, None],
                        lax.GatherDimensionNumbers(
                            offset_dims=(), collapsed_slice_dims=(0,),
                            start_index_map=(0,)),
                        (1,), mode=lax.GatherScatterMode.PROMISE_IN_BOUNDS)
                    for cc in range(H // 16):
                        sl = pl.ds(cc * 16, 16)
                        t = (ts[i, sl] + td[i, sl] - v[i, sl]) * iv
                        ts[i, sl] = jnp.maximum(t + bee_r[cc], 0.0)
                    return 0

                lax.fori_loop(0, 16, rowf, 0)
                return 0

            lax.fori_loop(0, ngrp, grp, 0)

        def outputs(step, S):
            ia, idd, ts, _, _, _, _ = sets[S]
            base = tb + step * CP
            if write_rp:
                pltpu.sync_copy(ts, rp_h.at[pl.ds(base, CP)])
            pltpu.sync_copy(ts, acc.at[ia], add=True)
            pltpu.sync_copy(ts, acc.at[idd], add=True)

        issue(0, 0)

        def pair(j, _):
            issue(2 * j + 1, 1)
            waits(0)
            compute(0, CP // 16)
            outputs(2 * j, 0)

            @pl.when(2 * j + 2 < NFULL)
            def _():
                issue(2 * j + 2, 0)

            waits(1)
            compute(1, CP // 16)
            outputs(2 * j + 1, 1)
            return 0

        lax.fori_loop(0, NPAIR, pair, 0)

        base = tb + NFULL * CP
        pltpu.sync_copy(src_h.at[pl.ds(base, TAIL)], iat)
        pltpu.sync_copy(dst_h.at[pl.ds(base, TAIL)], idt)
        pltpu.async_copy(t_h.at[iat], ts0.at[pl.ds(0, TAIL)], sem0)
        pltpu.async_copy(t_h.at[idt], td0.at[pl.ds(0, TAIL)], sem0)
        pltpu.sync_copy(u2_h.at[pl.ds(base, TAIL)], v0.at[pl.ds(0, TAIL)])
        pltpu.sync_copy(inv_h.at[pl.ds(base, TAIL)], iv0.at[pl.ds(0, TAIL)])
        pltpu.make_async_copy(t_h.at[pl.ds(0, TAIL)],
                              ts0.at[pl.ds(0, TAIL)], sem0).wait()
        pltpu.make_async_copy(t_h.at[pl.ds(0, TAIL)],
                              td0.at[pl.ds(0, TAIL)], sem0).wait()
        iv16 = iv0[pl.ds(0, 16)]
        for r in range(TAIL):
            iv = lax.gather(
                iv16, (zi16 + r)[:, None],
                lax.GatherDimensionNumbers(
                    offset_dims=(), collapsed_slice_dims=(0,),
                    start_index_map=(0,)),
                (1,), mode=lax.GatherScatterMode.PROMISE_IN_BOUNDS)
            for cc in range(H // 16):
                sl = pl.ds(cc * 16, 16)
                t = (ts0[r, sl] + td0[r, sl] - v0[r, sl]) * iv
                ts0[r, sl] = jnp.maximum(t + bee_r[cc], 0.0)
        if write_rp:
            pltpu.sync_copy(ts0.at[pl.ds(0, TAIL)], rp_h.at[pl.ds(base, TAIL)])
        pltpu.sync_copy(ts0.at[pl.ds(0, TAIL)], acc.at[iat], add=True)
        pltpu.sync_copy(ts0.at[pl.ds(0, TAIL)], acc.at[idt], add=True)

        plsc.subcore_barrier()
        _stripe_dump(acc, agg_h)

    return functools.partial(
        pl.kernel,
        out_type=tuple(outs),
        mesh=_mesh,
        scratch_types=(
            [pltpu.VMEM((CP,), jnp.int32)] * 4
            + [pltpu.VMEM((TAIL,), jnp.int32)] * 2
            + [pltpu.VMEM((CP, H), _f32)] * 6
            + [pltpu.VMEM((CP,), _f32)] * 2
            + [pltpu.VMEM_SHARED((NP, H), _f32),
               pltpu.SemaphoreType.DMA,
               pltpu.SemaphoreType.DMA]
        ),
    )(body)


_sc_edge_ref_rp = _make_edge_ref(True)
_sc_edge_ref_last = _make_edge_ref(False)


def _pq_body(h_ref, wa_ref, wb_ref, ca_ref, p_ref, q_ref):
    h = h_ref[...]
    p_ref[...] = jnp.dot(h, wa_ref[...], preferred_element_type=_f32) + ca_ref[...]
    q_ref[...] = jnp.dot(h, wb_ref[...], preferred_element_type=_f32)


def _tc_pq(h, wa, wb, ca):
    bn = 1000
    grid = NN // bn
    return pl.pallas_call(
        _pq_body,
        grid=(grid,),
        in_specs=[
            pl.BlockSpec((bn, H), lambda i: (i, 0)),
            pl.BlockSpec((H, H), lambda i: (0, 0)),
            pl.BlockSpec((H, H), lambda i: (0, 0)),
            pl.BlockSpec((1, H), lambda i: (0, 0)),
        ],
        out_specs=[
            pl.BlockSpec((bn, H), lambda i: (i, 0)),
            pl.BlockSpec((bn, H), lambda i: (i, 0)),
        ],
        out_shape=[jax.ShapeDtypeStruct((NN, H), _f32),
                   jax.ShapeDtypeStruct((NN, H), _f32)],
    )(h, wa, wb, ca)


def _make_mm(scale):
    def body(a_ref, w_ref, o_ref):
        o = jnp.dot(a_ref[...], w_ref[...], preferred_element_type=_f32)
        o_ref[...] = o * scale if scale != 1.0 else o

    def call(a, w):
        bm = 3200
        grid = EE // bm
        return pl.pallas_call(
            body,
            grid=(grid,),
            in_specs=[
                pl.BlockSpec((bm, H), lambda i: (i, 0)),
                pl.BlockSpec((H, H), lambda i: (0, 0)),
            ],
            out_specs=pl.BlockSpec((bm, H), lambda i: (i, 0)),
            out_shape=jax.ShapeDtypeStruct((EE, H), _f32),
        )(a, w)

    return call


_tc_mm = _make_mm(1.0)
_tc_mm2 = _make_mm(2.0)


def _heb_body(rp_ref, he_ref, w_ref, o_ref):
    hb = rp_ref[...] + he_ref[...]
    o_ref[...] = jnp.dot(hb, w_ref[...], preferred_element_type=_f32)


def _tc_heb_mm(rp, he, w):
    bm = 3200
    grid = EE // bm
    return pl.pallas_call(
        _heb_body,
        grid=(grid,),
        in_specs=[
            pl.BlockSpec((bm, H), lambda i: (i, 0)),
            pl.BlockSpec((bm, H), lambda i: (i, 0)),
            pl.BlockSpec((H, H), lambda i: (0, 0)),
        ],
        out_specs=pl.BlockSpec((bm, H), lambda i: (i, 0)),
        out_shape=jax.ShapeDtypeStruct((EE, H), _f32),
    )(rp, he, w)


def _t_body(s_ref, w_ref, o_ref):
    s = s_ref[0] + s_ref[1]
    o_ref[...] = jnp.dot(s, w_ref[...], preferred_element_type=_f32)


def _tc_t(s_part, w):
    bn = 1000
    grid = NN // bn
    return pl.pallas_call(
        _t_body,
        grid=(grid,),
        in_specs=[
            pl.BlockSpec((NC, bn, H), lambda i: (0, i, 0)),
            pl.BlockSpec((H, H), lambda i: (0, 0)),
        ],
        out_specs=pl.BlockSpec((bn, H), lambda i: (i, 0)),
        out_shape=jax.ShapeDtypeStruct((NN, H), _f32),
    )(s_part, w)


def _h_body(h_ref, s_ref, a_ref, w1_ref, w2_ref, b_ref, o_ref):
    agg = s_ref[0] + s_ref[1] + a_ref[0] + a_ref[1]
    o = (jnp.dot(h_ref[...], w1_ref[...], preferred_element_type=_f32)
         + jnp.dot(agg, w2_ref[...], preferred_element_type=_f32)
         + b_ref[...])
    o_ref[...] = jnp.maximum(o, 0.0)


def _tc_h(h, s_part, a_part, w1, w2, b):
    bn = 1000
    grid = NN // bn
    return pl.pallas_call(
        _h_body,
        grid=(grid,),
        in_specs=[
            pl.BlockSpec((bn, H), lambda i: (i, 0)),
            pl.BlockSpec((NC, bn, H), lambda i: (0, i, 0)),
            pl.BlockSpec((NC, bn, H), lambda i: (0, i, 0)),
            pl.BlockSpec((H, H), lambda i: (0, 0)),
            pl.BlockSpec((H, H), lambda i: (0, 0)),
            pl.BlockSpec((1, H), lambda i: (0, 0)),
        ],
        out_specs=pl.BlockSpec((bn, H), lambda i: (i, 0)),
        out_shape=jax.ShapeDtypeStruct((NN, H), _f32),
    )(h, s_part, a_part, w1, w2, b)


def _fc_body(h_ref, w_ref, b_ref, o_ref):
    o = jnp.dot(h_ref[...], w_ref[...], preferred_element_type=_f32) + b_ref[...]
    o_ref[...] = jnp.maximum(o, 0.0)


def _tc_fc(h, w, b):
    bn = 1000
    grid = NN // bn
    return pl.pallas_call(
        _fc_body,
        grid=(grid,),
        in_specs=[
            pl.BlockSpec((bn, H), lambda i: (i, 0)),
            pl.BlockSpec((H, H), lambda i: (0, 0)),
            pl.BlockSpec((1, H), lambda i: (0, 0)),
        ],
        out_specs=pl.BlockSpec((bn, H), lambda i: (i, 0)),
        out_shape=jax.ShapeDtypeStruct((NN, H), _f32),
    )(h, w, b)


def kernel(x, edge_index, W_ne0, b_ne0, W_ee0, b_ee0, W_en0, b_en0,
           W_ne1, b_ne1, W_ee1, b_ee1, W_en1, b_en1,
           W_ne2, b_ne2, W_ee2, b_ee2, W_en2, b_en2, W_fc, b_fc):
    src = edge_index[0]
    dst = edge_index[1]
    inv, _ = _sc_degree(src, dst)

    layers = [
        (W_ne0, b_ne0, W_ee0, b_ee0, W_en0, b_en0),
        (W_ne1, b_ne1, W_ee1, b_ee1, W_en1, b_en1),
        (W_ne2, b_ne2, W_ee2, b_ee2, W_en2, b_en2),
    ]
    h = x
    Rm = None
    for l, (W_ne, b_ne, W_ee, b_ee, W_en, b_en) in enumerate(layers):
        Wa, Wb = W_ne[:H], W_ne[H:2 * H]
        ca = b_ne + (W_ne[2 * H] if l == 0 else 0.0)
        P, Q = _tc_pq(h, Wa, Wb, ca.reshape(1, H))
        if l == 0:
            he, s_part = _sc_edge_up0(P, Q, src, dst)
        else:
            he, s_part = _sc_edge_up1(P, Q, Rm, src, dst)
        T = _tc_t(s_part, W_ee)
        U2 = _tc_mm2(he, W_ee)
        if l < 2:
            rp, a_part = _sc_edge_ref_rp(T, U2, inv, b_ee, src, dst)
            Wc_next = layers[l + 1][0][2 * H:]
            Rm = _tc_heb_mm(rp, he, Wc_next)
        else:
            (a_part,) = _sc_edge_ref_last(T, U2, inv, b_ee, src, dst)
        h = _tc_h(h, s_part, a_part, W_en[:H], W_en[H:], b_en.reshape(1, H))
    return _tc_fc(h, W_fc, b_fc.reshape(1, H))


# speedup vs baseline: 1.0649x; 1.0649x over previous
import functools
import jax
import jax.numpy as jnp
from jax import lax
from jax.experimental import pallas as pl
from jax.experimental.pallas import tpu as pltpu
from jax.experimental.pallas import tpu_sc as plsc

NN = 10000
EE = 160000
H = 128
NC = 2
NS = 16
NW = NC * NS
CB = 128
NCHUNK = EE // CB
KMAX = -(-NCHUNK // NW)
CBR = 64
NCHUNK_R = EE // CBR
KMAX_R = -(-NCHUNK_R // NW)
KMAX1 = -(-NCHUNK // NS)
RPT = 632
NP = RPT * NS

_mesh = plsc.VectorSubcoreMesh(
    core_axis_name="c", subcore_axis_name="s", num_cores=NC, num_subcores=NS)

_f32 = jnp.float32


def _wid():
    return lax.axis_index("s") * NC + lax.axis_index("c")


def _zero_fill(buf, rows):
    z = jnp.zeros((16,), _f32)

    def row(i, _):
        for cc in range(buf.shape[1] // 16):
            buf[i, pl.ds(cc * 16, 16)] = z
        return 0

    lax.fori_loop(0, rows, row, 0)


def _stripe_init(buf, acc):
    s = lax.axis_index("s")
    nb = buf.shape[0]
    _zero_fill(buf, nb)
    base = s * RPT
    for j in range(RPT // nb):
        pltpu.sync_copy(buf, acc.at[pl.ds(base + j * nb, nb)])
    rem = RPT % nb
    if rem:
        pltpu.sync_copy(buf.at[pl.ds(0, rem)],
                        acc.at[pl.ds(base + RPT - rem, rem)])


def _stripe_dump(acc, out):
    c = lax.axis_index("c")
    s = lax.axis_index("s")
    base = s * RPT
    for j in range(RPT // CB):
        pltpu.sync_copy(acc.at[pl.ds(base + j * CB, CB)],
                        out.at[c, pl.ds(base + j * CB, CB)])
    rem = RPT % CB
    if rem:
        pltpu.sync_copy(acc.at[pl.ds(base + RPT - rem, rem)],
                        out.at[c, pl.ds(base + RPT - rem, rem)])


@functools.partial(
    pl.kernel,
    out_type=(jax.ShapeDtypeStruct((EE,), _f32),
              jax.ShapeDtypeStruct((NC * NP, H), _f32)),
    mesh=_mesh,
    scratch_types=[
        pltpu.VMEM((CB,), jnp.int32),
        pltpu.VMEM((CB,), jnp.int32),
        pltpu.VMEM((CB,), jnp.int32),
        pltpu.VMEM((CB, H), _f32),
        pltpu.VMEM((CB, H), _f32),
        pltpu.VMEM((CB, H), _f32),
        pltpu.VMEM((CB,), _f32),
        pltpu.VMEM_SHARED((NP, H), _f32),
        pltpu.SemaphoreType.DMA,
        pltpu.SemaphoreType.DMA,
    ],
)
def _sc_degree(src_h, dst_h, inv_h, cnt_h, idx_s, idx_d, idx2, ones_b, buf_a,
               buf_b, buf_o, cnt, sem_a, sem_b):
    c = lax.axis_index("c")
    s = lax.axis_index("s")
    wid = _wid()
    one = jnp.ones((16,), _f32)

    def fill(i, _):
        for cc in range(H // 16):
            ones_b[i, pl.ds(cc * 16, 16)] = one
        return 0

    lax.fori_loop(0, CB, fill, 0)
    _stripe_init(buf_a, cnt)
    plsc.subcore_barrier()

    def count_step(k, _):
        cid = k * NS + s

        @pl.when(cid < NCHUNK)
        def _():
            base = pl.multiple_of(cid * CB, CB)
            pltpu.sync_copy(src_h.at[pl.ds(base, CB)], idx_s)
            pltpu.sync_copy(dst_h.at[pl.ds(base, CB)], idx_d)
            pltpu.sync_copy(ones_b, cnt.at[idx_s], add=True)
            pltpu.sync_copy(ones_b, cnt.at[idx_d], add=True)

        return 0

    lax.fori_loop(0, KMAX1, count_step, 0)
    plsc.subcore_barrier()

    for j in range(RPT // CB):
        pltpu.sync_copy(cnt.at[pl.ds(s * RPT + j * CB, CB)],
                        cnt_h.at[pl.ds(c * NP + s * RPT + j * CB, CB)])
    rem = RPT % CB
    pltpu.sync_copy(cnt.at[pl.ds(s * RPT + RPT - rem, rem)],
                    cnt_h.at[pl.ds(c * NP + s * RPT + RPT - rem, rem)])
    plsc.subcore_barrier()

    lane = lax.iota(jnp.int32, 16)
    off = jnp.full((16,), NP, jnp.int32) * c

    def inv_step(k, _):
        cid = k * NW + wid

        @pl.when(cid < NCHUNK)
        def _():
            base = pl.multiple_of(cid * CB, CB)
            pltpu.sync_copy(src_h.at[pl.ds(base, CB)], idx_s)
            pltpu.sync_copy(dst_h.at[pl.ds(base, CB)], idx_d)

            def addoff(src_ref):
                def go(g, _):
                    sl = pl.ds(g * 16, 16)
                    idx2[sl] = src_ref[sl] + off
                    return 0

                lax.fori_loop(0, CB // 16, go, 0)

            addoff(idx_s)
            cpa = pltpu.async_copy(cnt_h.at[idx2], buf_a, sem_a)
            cpa.wait()
            addoff(idx_d)
            cpb = pltpu.async_copy(cnt_h.at[idx2], buf_b, sem_b)
            cpb.wait()

            def grp(g, _):
                def rowf(r, acc):
                    j = g * 16 + r
                    a = buf_a[j, pl.ds(0, 16)]
                    b = buf_b[j, pl.ds(0, 16)]
                    iv = 1.0 / jnp.maximum(a + b - 2.0, 1.0)
                    return jnp.where(lane == r, iv, acc)

                buf_o[pl.ds(g * 16, 16)] = lax.fori_loop(
                    0, 16, rowf, jnp.zeros((16,), _f32))
                return 0

            lax.fori_loop(0, CB // 16, grp, 0)
            pltpu.sync_copy(buf_o, inv_h.at[pl.ds(base, CB)])

        return 0

    lax.fori_loop(0, KMAX, inv_step, 0)


EPT = EE // NW
CP = 64
NFULL = EPT // CP
TAIL = EPT - NFULL * CP
NPAIR = NFULL // 2


def _make_edge_up(has_r):
    scratch = (
        [pltpu.VMEM((CP,), jnp.int32)] * 4
        + [pltpu.VMEM((TAIL,), jnp.int32)] * 2
        + [pltpu.VMEM((CP, H), _f32)] * (5 if has_r else 4)
        + [pltpu.VMEM_SHARED((NP, H), _f32),
           pltpu.SemaphoreType.DMA,
           pltpu.SemaphoreType.DMA]
    )

    def body(p_h, q_h, *rest):
        if has_r:
            (r_h, src_h, dst_h, he_h, s_h, ia0, id0, ia1, id1, iat, idt,
             a0, b0, a1, b1, c, acc, sem0, sem1) = rest
        else:
            (src_h, dst_h, he_h, s_h, ia0, id0, ia1, id1, iat, idt,
             a0, b0, a1, b1, acc, sem0, sem1) = rest
            c = None
        wid = _wid()
        tb = wid * EPT
        _stripe_init(a0, acc)
        plsc.subcore_barrier()
        sets = [(ia0, id0, a0, b0, sem0), (ia1, id1, a1, b1, sem1)]

        def issue(step, S):
            ia, idd, a, b, sem = sets[S]
            base = tb + step * CP
            pltpu.sync_copy(src_h.at[pl.ds(base, CP)], ia)
            pltpu.sync_copy(dst_h.at[pl.ds(base, CP)], idd)
            pltpu.async_copy(p_h.at[ia], a, sem)
            pltpu.async_copy(q_h.at[idd], b, sem)

        def process(step, S, nrows):
            ia, idd, a, b, sem = sets[S]
            base = tb + step * CP
            if has_r:
                pltpu.async_copy(r_h.at[pl.ds(base, nrows)],
                                 c.at[pl.ds(0, nrows)], sem)
            pltpu.make_async_copy(p_h.at[pl.ds(0, nrows)],
                                  a.at[pl.ds(0, nrows)], sem).wait()
            pltpu.make_async_copy(p_h.at[pl.ds(0, nrows)],
                                  b.at[pl.ds(0, nrows)], sem).wait()
            if has_r:
                pltpu.make_async_copy(r_h.at[pl.ds(0, nrows)],
                                      c.at[pl.ds(0, nrows)], sem).wait()

            def row(i, _):
                for cc in range(H // 16):
                    sl = pl.ds(cc * 16, 16)
                    v = a[i, sl] + b[i, sl]
                    if has_r:
                        v = v + c[i, sl]
                    a[i, sl] = jnp.maximum(v, 0.0)
                return 0

            lax.fori_loop(0, nrows, row, 0)
            pltpu.sync_copy(a.at[pl.ds(0, nrows)], he_h.at[pl.ds(base, nrows)])

        def scat(idx_ref, buf, nrows):
            pltpu.sync_copy(buf.at[pl.ds(0, nrows)], acc.at[idx_ref], add=True)

        issue(0, 0)

        def pair(j, _):
            issue(2 * j + 1, 1)
            process(2 * j, 0, CP)
            scat(ia0, a0, CP)
            scat(id0, a0, CP)

            @pl.when(2 * j + 2 < NFULL)
            def _():
                issue(2 * j + 2, 0)

            process(2 * j + 1, 1, CP)
            scat(ia1, a1, CP)
            scat(id1, a1, CP)
            return 0

        lax.fori_loop(0, NPAIR, pair, 0)

        base = tb + NFULL * CP
        pltpu.sync_copy(src_h.at[pl.ds(base, TAIL)], iat)
        pltpu.sync_copy(dst_h.at[pl.ds(base, TAIL)], idt)
        pltpu.async_copy(p_h.at[iat], a0.at[pl.ds(0, TAIL)], sem0)
        pltpu.async_copy(q_h.at[idt], b0.at[pl.ds(0, TAIL)], sem0)
        process(NFULL, 0, TAIL)
        scat(iat, a0, TAIL)
        scat(idt, a0, TAIL)

        plsc.subcore_barrier()
        _stripe_dump(acc, s_h)

    return functools.partial(
        pl.kernel,
        out_type=(jax.ShapeDtypeStruct((EE, H), _f32),
                  jax.ShapeDtypeStruct((NC, NP, H), _f32)),
        mesh=_mesh,
        scratch_types=scratch,
    )(body)


_sc_edge_up0 = _make_edge_up(False)
_sc_edge_up1 = _make_edge_up(True)


def _make_edge_ref(write_rp):
    outs = [jax.ShapeDtypeStruct((NC, NP, H), _f32)]
    if write_rp:
        outs = [jax.ShapeDtypeStruct((EE, H), _f32)] + outs

    def body(t_h, u2_h, inv_h, bee_h, src_h, dst_h, *rest):
        if write_rp:
            (rp_h, agg_h, ia0, id0, ia1, id1, iat, idt,
             ts0, td0, ts1, td1, v, iv_s,
             acc, sem0, sem1) = rest
        else:
            (agg_h, ia0, id0, ia1, id1, iat, idt,
             ts0, td0, ts1, td1, v, iv_s,
             acc, sem0, sem1) = rest
        wid = _wid()
        tb = wid * EPT
        pltpu.sync_copy(bee_h, td0.at[0])
        bee_r = [td0[0, pl.ds(cc * 16, 16)] for cc in range(H // 16)]
        _stripe_init(ts0, acc)
        plsc.subcore_barrier()
        zi16 = jnp.zeros((16,), jnp.int32)
        sets = [(ia0, id0, ts0, td0, sem0),
                (ia1, id1, ts1, td1, sem1)]

        def issue(step, S):
            ia, idd, ts, td, sem = sets[S]
            base = tb + step * CP
            pltpu.sync_copy(src_h.at[pl.ds(base, CP)], ia)
            pltpu.sync_copy(dst_h.at[pl.ds(base, CP)], idd)
            pltpu.async_copy(t_h.at[ia], ts, sem)
            pltpu.async_copy(t_h.at[idd], td, sem)

        def waits(S):
            _, _, ts, td, sem = sets[S]
            pltpu.make_async_copy(t_h.at[pl.ds(0, CP)], ts, sem).wait()
            pltpu.make_async_copy(t_h.at[pl.ds(0, CP)], td, sem).wait()

        def compute(step, S, ngrp):
            _, _, ts, td, _ = sets[S]
            base = tb + step * CP
            pltpu.sync_copy(u2_h.at[pl.ds(base, 16 * ngrp)],
                            v.at[pl.ds(0, 16 * ngrp)])
            pltpu.sync_copy(inv_h.at[pl.ds(base, 16 * ngrp)],
                            iv_s.at[pl.ds(0, 16 * ngrp)])

            def grp(g, _):
                iv16 = iv_s[pl.ds(g * 16, 16)]

                def rowf(r, _):
                    i = g * 16 + r
                    iv = lax.gather(
                        iv16, (zi16 + r)[:, None],
                        lax.GatherDimensionNumbers(
                            offset_dims=(), collapsed_slice_dims=(0,),
                            start_index_map=(0,)),
                        (1,), mode=lax.GatherScatterMode.PROMISE_IN_BOUNDS)
                    for cc in range(H // 16):
                        sl = pl.ds(cc * 16, 16)
                        t = (ts[i, sl] + td[i, sl] - v[i, sl]) * iv
                        ts[i, sl] = jnp.maximum(t + bee_r[cc], 0.0)
                    return 0

                lax.fori_loop(0, 16, rowf, 0)
                return 0

            lax.fori_loop(0, ngrp, grp, 0)

        def outputs(step, S):
            ia, idd, ts, _, _ = sets[S]
            base = tb + step * CP
            if write_rp:
                pltpu.sync_copy(ts, rp_h.at[pl.ds(base, CP)])
            pltpu.sync_copy(ts, acc.at[ia], add=True)
            pltpu.sync_copy(ts, acc.at[idd], add=True)

        issue(0, 0)

        def pair(j, _):
            issue(2 * j + 1, 1)
            waits(0)
            compute(2 * j, 0, CP // 16)
            outputs(2 * j, 0)

            @pl.when(2 * j + 2 < NFULL)
            def _():
                issue(2 * j + 2, 0)

            waits(1)
            compute(2 * j + 1, 1, CP // 16)
            outputs(2 * j + 1, 1)
            return 0

        lax.fori_loop(0, NPAIR, pair, 0)

        base = tb + NFULL * CP
        pltpu.sync_copy(src_h.at[pl.ds(base, TAIL)], iat)
        pltpu.sync_copy(dst_h.at[pl.ds(base, TAIL)], idt)
        pltpu.async_copy(t_h.at[iat], ts0.at[pl.ds(0, TAIL)], sem0)
        pltpu.async_copy(t_h.at[idt], td0.at[pl.ds(0, TAIL)], sem0)
        pltpu.sync_copy(u2_h.at[pl.ds(base, TAIL)], v.at[pl.ds(0, TAIL)])
        pltpu.sync_copy(inv_h.at[pl.ds(base, TAIL)], iv_s.at[pl.ds(0, TAIL)])
        pltpu.make_async_copy(t_h.at[pl.ds(0, TAIL)],
                              ts0.at[pl.ds(0, TAIL)], sem0).wait()
        pltpu.make_async_copy(t_h.at[pl.ds(0, TAIL)],
                              td0.at[pl.ds(0, TAIL)], sem0).wait()
        iv16 = iv_s[pl.ds(0, 16)]
        for r in range(TAIL):
            iv = lax.gather(
                iv16, (zi16 + r)[:, None],
                lax.GatherDimensionNumbers(
                    offset_dims=(), collapsed_slice_dims=(0,),
                    start_index_map=(0,)),
                (1,), mode=lax.GatherScatterMode.PROMISE_IN_BOUNDS)
            for cc in range(H // 16):
                sl = pl.ds(cc * 16, 16)
                t = (ts0[r, sl] + td0[r, sl] - v[r, sl]) * iv
                ts0[r, sl] = jnp.maximum(t + bee_r[cc], 0.0)
        if write_rp:
            pltpu.sync_copy(ts0.at[pl.ds(0, TAIL)], rp_h.at[pl.ds(base, TAIL)])
        pltpu.sync_copy(ts0.at[pl.ds(0, TAIL)], acc.at[iat], add=True)
        pltpu.sync_copy(ts0.at[pl.ds(0, TAIL)], acc.at[idt], add=True)

        plsc.subcore_barrier()
        _stripe_dump(acc, agg_h)

    return functools.partial(
        pl.kernel,
        out_type=tuple(outs),
        mesh=_mesh,
        scratch_types=(
            [pltpu.VMEM((CP,), jnp.int32)] * 4
            + [pltpu.VMEM((TAIL,), jnp.int32)] * 2
            + [pltpu.VMEM((CP, H), _f32)] * 5
            + [pltpu.VMEM((CP,), _f32)]
            + [pltpu.VMEM_SHARED((NP, H), _f32),
               pltpu.SemaphoreType.DMA,
               pltpu.SemaphoreType.DMA]
        ),
    )(body)


_sc_edge_ref_rp = _make_edge_ref(True)
_sc_edge_ref_last = _make_edge_ref(False)


def _pq_body(h_ref, wa_ref, wb_ref, ca_ref, p_ref, q_ref):
    h = h_ref[...]
    p_ref[...] = jnp.dot(h, wa_ref[...], preferred_element_type=_f32) + ca_ref[...]
    q_ref[...] = jnp.dot(h, wb_ref[...], preferred_element_type=_f32)


def _tc_pq(h, wa, wb, ca):
    bn = 1000
    grid = NN // bn
    return pl.pallas_call(
        _pq_body,
        grid=(grid,),
        in_specs=[
            pl.BlockSpec((bn, H), lambda i: (i, 0)),
            pl.BlockSpec((H, H), lambda i: (0, 0)),
            pl.BlockSpec((H, H), lambda i: (0, 0)),
            pl.BlockSpec((1, H), lambda i: (0, 0)),
        ],
        out_specs=[
            pl.BlockSpec((bn, H), lambda i: (i, 0)),
            pl.BlockSpec((bn, H), lambda i: (i, 0)),
        ],
        out_shape=[jax.ShapeDtypeStruct((NN, H), _f32),
                   jax.ShapeDtypeStruct((NN, H), _f32)],
    )(h, wa, wb, ca)


def _make_mm(scale):
    def body(a_ref, w_ref, o_ref):
        o = jnp.dot(a_ref[...], w_ref[...], preferred_element_type=_f32)
        o_ref[...] = o * scale if scale != 1.0 else o

    def call(a, w):
        bm = 3200
        grid = EE // bm
        return pl.pallas_call(
            body,
            grid=(grid,),
            in_specs=[
                pl.BlockSpec((bm, H), lambda i: (i, 0)),
                pl.BlockSpec((H, H), lambda i: (0, 0)),
            ],
            out_specs=pl.BlockSpec((bm, H), lambda i: (i, 0)),
            out_shape=jax.ShapeDtypeStruct((EE, H), _f32),
        )(a, w)

    return call


_tc_mm = _make_mm(1.0)
_tc_mm2 = _make_mm(2.0)


def _heb_body(rp_ref, he_ref, w_ref, o_ref):
    hb = rp_ref[...] + he_ref[...]
    o_ref[...] = jnp.dot(hb, w_ref[...], preferred_element_type=_f32)


def _tc_heb_mm(rp, he, w):
    bm = 3200
    grid = EE // bm
    return pl.pallas_call(
        _heb_body,
        grid=(grid,),
        in_specs=[
            pl.BlockSpec((bm, H), lambda i: (i, 0)),
            pl.BlockSpec((bm, H), lambda i: (i, 0)),
            pl.BlockSpec((H, H), lambda i: (0, 0)),
        ],
        out_specs=pl.BlockSpec((bm, H), lambda i: (i, 0)),
        out_shape=jax.ShapeDtypeStruct((EE, H), _f32),
    )(rp, he, w)


def _t_body(s_ref, w_ref, o_ref):
    s = s_ref[0] + s_ref[1]
    o_ref[...] = jnp.dot(s, w_ref[...], preferred_element_type=_f32)


def _tc_t(s_part, w):
    bn = 1000
    grid = NN // bn
    return pl.pallas_call(
        _t_body,
        grid=(grid,),
        in_specs=[
            pl.BlockSpec((NC, bn, H), lambda i: (0, i, 0)),
            pl.BlockSpec((H, H), lambda i: (0, 0)),
        ],
        out_specs=pl.BlockSpec((bn, H), lambda i: (i, 0)),
        out_shape=jax.ShapeDtypeStruct((NN, H), _f32),
    )(s_part, w)


def _h_body(h_ref, s_ref, a_ref, w1_ref, w2_ref, b_ref, o_ref):
    agg = s_ref[0] + s_ref[1] + a_ref[0] + a_ref[1]
    o = (jnp.dot(h_ref[...], w1_ref[...], preferred_element_type=_f32)
         + jnp.dot(agg, w2_ref[...], preferred_element_type=_f32)
         + b_ref[...])
    o_ref[...] = jnp.maximum(o, 0.0)


def _tc_h(h, s_part, a_part, w1, w2, b):
    bn = 1000
    grid = NN // bn
    return pl.pallas_call(
        _h_body,
        grid=(grid,),
        in_specs=[
            pl.BlockSpec((bn, H), lambda i: (i, 0)),
            pl.BlockSpec((NC, bn, H), lambda i: (0, i, 0)),
            pl.BlockSpec((NC, bn, H), lambda i: (0, i, 0)),
            pl.BlockSpec((H, H), lambda i: (0, 0)),
            pl.BlockSpec((H, H), lambda i: (0, 0)),
            pl.BlockSpec((1, H), lambda i: (0, 0)),
        ],
        out_specs=pl.BlockSpec((bn, H), lambda i: (i, 0)),
        out_shape=jax.ShapeDtypeStruct((NN, H), _f32),
    )(h, s_part, a_part, w1, w2, b)


def _fc_body(h_ref, w_ref, b_ref, o_ref):
    o = jnp.dot(h_ref[...], w_ref[...], preferred_element_type=_f32) + b_ref[...]
    o_ref[...] = jnp.maximum(o, 0.0)


def _tc_fc(h, w, b):
    bn = 1000
    grid = NN // bn
    return pl.pallas_call(
        _fc_body,
        grid=(grid,),
        in_specs=[
            pl.BlockSpec((bn, H), lambda i: (i, 0)),
            pl.BlockSpec((H, H), lambda i: (0, 0)),
            pl.BlockSpec((1, H), lambda i: (0, 0)),
        ],
        out_specs=pl.BlockSpec((bn, H), lambda i: (i, 0)),
        out_shape=jax.ShapeDtypeStruct((NN, H), _f32),
    )(h, w, b)


def kernel(x, edge_index, W_ne0, b_ne0, W_ee0, b_ee0, W_en0, b_en0,
           W_ne1, b_ne1, W_ee1, b_ee1, W_en1, b_en1,
           W_ne2, b_ne2, W_ee2, b_ee2, W_en2, b_en2, W_fc, b_fc):
    src = edge_index[0]
    dst = edge_index[1]
    inv, _ = _sc_degree(src, dst)

    layers = [
        (W_ne0, b_ne0, W_ee0, b_ee0, W_en0, b_en0),
        (W_ne1, b_ne1, W_ee1, b_ee1, W_en1, b_en1),
        (W_ne2, b_ne2, W_ee2, b_ee2, W_en2, b_en2),
    ]
    h = x
    Rm = None
    for l, (W_ne, b_ne, W_ee, b_ee, W_en, b_en) in enumerate(layers):
        Wa, Wb = W_ne[:H], W_ne[H:2 * H]
        ca = b_ne + (W_ne[2 * H] if l == 0 else 0.0)
        P, Q = _tc_pq(h, Wa, Wb, ca.reshape(1, H))
        if l == 0:
            he, s_part = _sc_edge_up0(P, Q, src, dst)
        else:
            he, s_part = _sc_edge_up1(P, Q, Rm, src, dst)
        T = _tc_t(s_part, W_ee)
        U2 = _tc_mm2(he, W_ee)
        if l < 2:
            rp, a_part = _sc_edge_ref_rp(T, U2, inv, b_ee, src, dst)
            Wc_next = layers[l + 1][0][2 * H:]
            Rm = _tc_heb_mm(rp, he, Wc_next)
        else:
            (a_part,) = _sc_edge_ref_last(T, U2, inv, b_ee, src, dst)
        h = _tc_h(h, s_part, a_part, W_en[:H], W_en[H:], b_en.reshape(1, H))
    return _tc_fc(h, W_fc, b_fc.reshape(1, H))


# speedup vs baseline: 1.1211x; 1.0528x over previous
import functools
import jax
import jax.numpy as jnp
from jax import lax
from jax.experimental import pallas as pl
from jax.experimental.pallas import tpu as pltpu
from jax.experimental.pallas import tpu_sc as plsc

NN = 10000
EE = 160000
H = 128
NC = 2
NS = 16
NW = NC * NS
CB = 128
NCHUNK = EE // CB
KMAX = -(-NCHUNK // NW)
CBR = 64
NCHUNK_R = EE // CBR
KMAX_R = -(-NCHUNK_R // NW)
KMAX1 = -(-NCHUNK // NS)
RPT = 632
NP = RPT * NS

_mesh = plsc.VectorSubcoreMesh(
    core_axis_name="c", subcore_axis_name="s", num_cores=NC, num_subcores=NS)

_f32 = jnp.float32


def _wid():
    return lax.axis_index("s") * NC + lax.axis_index("c")


def _zero_fill(buf, rows):
    z = jnp.zeros((16,), _f32)

    def row(i, _):
        for cc in range(buf.shape[1] // 16):
            buf[i, pl.ds(cc * 16, 16)] = z
        return 0

    lax.fori_loop(0, rows, row, 0)


def _stripe_init(buf, acc):
    s = lax.axis_index("s")
    nb = buf.shape[0]
    _zero_fill(buf, nb)
    base = s * RPT
    for j in range(RPT // nb):
        pltpu.sync_copy(buf, acc.at[pl.ds(base + j * nb, nb)])
    rem = RPT % nb
    if rem:
        pltpu.sync_copy(buf.at[pl.ds(0, rem)],
                        acc.at[pl.ds(base + RPT - rem, rem)])


def _stripe_dump(acc, out):
    c = lax.axis_index("c")
    s = lax.axis_index("s")
    base = s * RPT
    for j in range(RPT // CB):
        pltpu.sync_copy(acc.at[pl.ds(base + j * CB, CB)],
                        out.at[c, pl.ds(base + j * CB, CB)])
    rem = RPT % CB
    if rem:
        pltpu.sync_copy(acc.at[pl.ds(base + RPT - rem, rem)],
                        out.at[c, pl.ds(base + RPT - rem, rem)])


@functools.partial(
    pl.kernel,
    out_type=(jax.ShapeDtypeStruct((EE,), _f32),
              jax.ShapeDtypeStruct((NC * NP, H), _f32)),
    mesh=_mesh,
    scratch_types=[
        pltpu.VMEM((CB,), jnp.int32),
        pltpu.VMEM((CB,), jnp.int32),
        pltpu.VMEM((CB,), jnp.int32),
        pltpu.VMEM((CB,), jnp.int32),
        pltpu.VMEM((CB, H), _f32),
        pltpu.VMEM((CB, H), _f32),
        pltpu.VMEM((CB, H), _f32),
        pltpu.VMEM((CB,), _f32),
        pltpu.VMEM_SHARED((NP, H), _f32),
        pltpu.SemaphoreType.DMA,
        pltpu.SemaphoreType.DMA,
    ],
)
def _sc_degree(src_h, dst_h, inv_h, cnt_h, idx_s, idx_d, idx2, idx3, ones_b,
               buf_a, buf_b, buf_o, cnt, sem_a, sem_b):
    c = lax.axis_index("c")
    s = lax.axis_index("s")
    wid = _wid()
    one = jnp.ones((16,), _f32)

    def fill(i, _):
        for cc in range(H // 16):
            ones_b[i, pl.ds(cc * 16, 16)] = one
        return 0

    lax.fori_loop(0, CB, fill, 0)
    _stripe_init(buf_a, cnt)
    plsc.subcore_barrier()

    def count_step(k, _):
        cid = k * NS + s

        @pl.when(cid < NCHUNK)
        def _():
            base = pl.multiple_of(cid * CB, CB)
            pltpu.sync_copy(src_h.at[pl.ds(base, CB)], idx_s)
            pltpu.sync_copy(dst_h.at[pl.ds(base, CB)], idx_d)
            pltpu.sync_copy(ones_b, cnt.at[idx_s], add=True)
            pltpu.sync_copy(ones_b, cnt.at[idx_d], add=True)

        return 0

    lax.fori_loop(0, KMAX1, count_step, 0)
    plsc.subcore_barrier()

    for j in range(RPT // CB):
        pltpu.sync_copy(cnt.at[pl.ds(s * RPT + j * CB, CB)],
                        cnt_h.at[pl.ds(c * NP + s * RPT + j * CB, CB)])
    rem = RPT % CB
    pltpu.sync_copy(cnt.at[pl.ds(s * RPT + RPT - rem, rem)],
                    cnt_h.at[pl.ds(c * NP + s * RPT + RPT - rem, rem)])
    plsc.subcore_barrier()

    lane = lax.iota(jnp.int32, 16)
    off = jnp.full((16,), NP, jnp.int32) * c

    def inv_step(k, _):
        cid = k * NW + wid

        @pl.when(cid < NCHUNK)
        def _():
            base = pl.multiple_of(cid * CB, CB)
            pltpu.sync_copy(src_h.at[pl.ds(base, CB)], idx_s)
            pltpu.sync_copy(dst_h.at[pl.ds(base, CB)], idx_d)

            def addoff(src_ref, dst_ref):
                def go(g, _):
                    sl = pl.ds(g * 16, 16)
                    dst_ref[sl] = src_ref[sl] + off
                    return 0

                lax.fori_loop(0, CB // 16, go, 0)

            addoff(idx_s, idx2)
            addoff(idx_d, idx3)
            cpa = pltpu.async_copy(cnt_h.at[idx2], buf_a, sem_a)
            cpb = pltpu.async_copy(cnt_h.at[idx3], buf_b, sem_b)
            cpa.wait()
            cpb.wait()

            def grp(g, _):
                def rowf(r, acc):
                    j = g * 16 + r
                    a = buf_a[j, pl.ds(0, 16)]
                    b = buf_b[j, pl.ds(0, 16)]
                    iv = 1.0 / jnp.maximum(a + b - 2.0, 1.0)
                    return jnp.where(lane == r, iv, acc)

                buf_o[pl.ds(g * 16, 16)] = lax.fori_loop(
                    0, 16, rowf, jnp.zeros((16,), _f32))
                return 0

            lax.fori_loop(0, CB // 16, grp, 0)
            pltpu.sync_copy(buf_o, inv_h.at[pl.ds(base, CB)])

        return 0

    lax.fori_loop(0, KMAX, inv_step, 0)


def _make_edge_up(has_r):
    scratch = [
        pltpu.VMEM((CB,), jnp.int32),
        pltpu.VMEM((CB,), jnp.int32),
        pltpu.VMEM((CB, H), _f32),
        pltpu.VMEM((CB, H), _f32),
    ]
    if has_r:
        scratch.append(pltpu.VMEM((CB, H), _f32))
    scratch += [
        pltpu.VMEM_SHARED((NP, H), _f32),
        pltpu.SemaphoreType.DMA,
        pltpu.SemaphoreType.DMA,
    ]

    def body(p_h, q_h, *rest):
        if has_r:
            (r_h, src_h, dst_h, he_h, s_h,
             idx_s, idx_d, buf_a, buf_b, buf_c, acc, sem_a, sem_b) = rest
        else:
            (src_h, dst_h, he_h, s_h,
             idx_s, idx_d, buf_a, buf_b, acc, sem_a, sem_b) = rest
        wid = _wid()
        _stripe_init(buf_a, acc)
        plsc.subcore_barrier()

        def step(k, _):
            cid = k * NW + wid

            @pl.when(cid < NCHUNK)
            def _():
                base = pl.multiple_of(cid * CB, CB)
                pltpu.sync_copy(src_h.at[pl.ds(base, CB)], idx_s)
                pltpu.sync_copy(dst_h.at[pl.ds(base, CB)], idx_d)
                cpa = pltpu.async_copy(p_h.at[idx_s], buf_a, sem_a)
                cpb = pltpu.async_copy(q_h.at[idx_d], buf_b, sem_b)
                if has_r:
                    pltpu.sync_copy(r_h.at[pl.ds(base, CB)], buf_c)
                cpa.wait()
                cpb.wait()

                def row(i, _):
                    for cc in range(H // 16):
                        sl = pl.ds(cc * 16, 16)
                        v = buf_a[i, sl] + buf_b[i, sl]
                        if has_r:
                            v = v + buf_c[i, sl]
                        buf_a[i, sl] = jnp.maximum(v, 0.0)
                    return 0

                lax.fori_loop(0, CB, row, 0)
                pltpu.sync_copy(buf_a, he_h.at[pl.ds(base, CB)])
                pltpu.sync_copy(buf_a, acc.at[idx_s], add=True)
                pltpu.sync_copy(buf_a, acc.at[idx_d], add=True)

            return 0

        lax.fori_loop(0, KMAX, step, 0)
        plsc.subcore_barrier()
        _stripe_dump(acc, s_h)

    return functools.partial(
        pl.kernel,
        out_type=(jax.ShapeDtypeStruct((EE, H), _f32),
                  jax.ShapeDtypeStruct((NC, NP, H), _f32)),
        mesh=_mesh,
        scratch_types=scratch,
    )(body)


_sc_edge_up0 = _make_edge_up(False)
_sc_edge_up1 = _make_edge_up(True)


def _make_edge_ref(write_rp):
    outs = [jax.ShapeDtypeStruct((NC, NP, H), _f32)]
    if write_rp:
        outs = [jax.ShapeDtypeStruct((EE, H), _f32)] + outs

    def body(t_h, u2_h, inv_h, bee_h, src_h, dst_h, *rest):
        if write_rp:
            (rp_h, agg_h, idx_s, idx_d, buf_ts, buf_td, buf_v, buf_inv,
             bee_v, acc, sem_a, sem_b) = rest
        else:
            (agg_h, idx_s, idx_d, buf_ts, buf_td, buf_v, buf_inv,
             bee_v, acc, sem_a, sem_b) = rest
        wid = _wid()
        pltpu.sync_copy(bee_h, bee_v)
        _stripe_init(buf_ts, acc)
        plsc.subcore_barrier()

        bee_r = [bee_v[pl.ds(cc * 16, 16)] for cc in range(H // 16)]
        zi16 = jnp.zeros((16,), jnp.int32)

        def step(k, _):
            cid = k * NW + wid

            @pl.when(cid < NCHUNK)
            def _():
                base = pl.multiple_of(cid * CB, CB)
                pltpu.sync_copy(src_h.at[pl.ds(base, CB)], idx_s)
                pltpu.sync_copy(dst_h.at[pl.ds(base, CB)], idx_d)
                cpa = pltpu.async_copy(t_h.at[idx_s], buf_ts, sem_a)
                cpb = pltpu.async_copy(t_h.at[idx_d], buf_td, sem_b)
                pltpu.sync_copy(u2_h.at[pl.ds(base, CB)], buf_v)
                pltpu.sync_copy(inv_h.at[pl.ds(base, CB)], buf_inv)
                cpa.wait()
                cpb.wait()

                def grp(g, _):
                    iv16 = buf_inv[pl.ds(g * 16, 16)]

                    def rowf(r, _):
                        i = g * 16 + r
                        iv = lax.gather(
                            iv16, (zi16 + r)[:, None],
                            lax.GatherDimensionNumbers(
                                offset_dims=(), collapsed_slice_dims=(0,),
                                start_index_map=(0,)),
                            (1,), mode=lax.GatherScatterMode.PROMISE_IN_BOUNDS)
                        for cc in range(H // 16):
                            sl = pl.ds(cc * 16, 16)
                            t = (buf_ts[i, sl] + buf_td[i, sl]
                                 - buf_v[i, sl]) * iv
                            buf_ts[i, sl] = jnp.maximum(t + bee_r[cc], 0.0)
                        return 0

                    lax.fori_loop(0, 16, rowf, 0)
                    return 0

                lax.fori_loop(0, CB // 16, grp, 0)
                if write_rp:
                    pltpu.sync_copy(buf_ts, rp_h.at[pl.ds(base, CB)])
                pltpu.sync_copy(buf_ts, acc.at[idx_s], add=True)
                pltpu.sync_copy(buf_ts, acc.at[idx_d], add=True)

            return 0

        lax.fori_loop(0, KMAX, step, 0)
        plsc.subcore_barrier()
        _stripe_dump(acc, agg_h)

    return functools.partial(
        pl.kernel,
        out_type=tuple(outs),
        mesh=_mesh,
        scratch_types=[
            pltpu.VMEM((CB,), jnp.int32),
            pltpu.VMEM((CB,), jnp.int32),
            pltpu.VMEM((CB, H), _f32),
            pltpu.VMEM((CB, H), _f32),
            pltpu.VMEM((CB, H), _f32),
            pltpu.VMEM((CB,), _f32),
            pltpu.VMEM((H,), _f32),
            pltpu.VMEM_SHARED((NP, H), _f32),
            pltpu.SemaphoreType.DMA,
            pltpu.SemaphoreType.DMA,
        ],
    )(body)


_sc_edge_ref_rp = _make_edge_ref(True)
_sc_edge_ref_last = _make_edge_ref(False)


def _pq_body(h_ref, wa_ref, wb_ref, ca_ref, p_ref, q_ref):
    h = h_ref[...]
    p_ref[...] = jnp.dot(h, wa_ref[...], preferred_element_type=_f32) + ca_ref[...]
    q_ref[...] = jnp.dot(h, wb_ref[...], preferred_element_type=_f32)


def _tc_pq(h, wa, wb, ca):
    bn = 1000
    grid = NN // bn
    return pl.pallas_call(
        _pq_body,
        grid=(grid,),
        in_specs=[
            pl.BlockSpec((bn, H), lambda i: (i, 0)),
            pl.BlockSpec((H, H), lambda i: (0, 0)),
            pl.BlockSpec((H, H), lambda i: (0, 0)),
            pl.BlockSpec((1, H), lambda i: (0, 0)),
        ],
        out_specs=[
            pl.BlockSpec((bn, H), lambda i: (i, 0)),
            pl.BlockSpec((bn, H), lambda i: (i, 0)),
        ],
        out_shape=[jax.ShapeDtypeStruct((NN, H), _f32),
                   jax.ShapeDtypeStruct((NN, H), _f32)],
    )(h, wa, wb, ca)


def _make_mm(scale):
    def body(a_ref, w_ref, o_ref):
        o = jnp.dot(a_ref[...], w_ref[...], preferred_element_type=_f32)
        o_ref[...] = o * scale if scale != 1.0 else o

    def call(a, w):
        bm = 3200
        grid = EE // bm
        return pl.pallas_call(
            body,
            grid=(grid,),
            in_specs=[
                pl.BlockSpec((bm, H), lambda i: (i, 0)),
                pl.BlockSpec((H, H), lambda i: (0, 0)),
            ],
            out_specs=pl.BlockSpec((bm, H), lambda i: (i, 0)),
            out_shape=jax.ShapeDtypeStruct((EE, H), _f32),
        )(a, w)

    return call


_tc_mm = _make_mm(1.0)
_tc_mm2 = _make_mm(2.0)


def _heb_body(rp_ref, he_ref, w_ref, o_ref):
    hb = rp_ref[...] + he_ref[...]
    o_ref[...] = jnp.dot(hb, w_ref[...], preferred_element_type=_f32)


def _tc_heb_mm(rp, he, w):
    bm = 3200
    grid = EE // bm
    return pl.pallas_call(
        _heb_body,
        grid=(grid,),
        in_specs=[
            pl.BlockSpec((bm, H), lambda i: (i, 0)),
            pl.BlockSpec((bm, H), lambda i: (i, 0)),
            pl.BlockSpec((H, H), lambda i: (0, 0)),
        ],
        out_specs=pl.BlockSpec((bm, H), lambda i: (i, 0)),
        out_shape=jax.ShapeDtypeStruct((EE, H), _f32),
    )(rp, he, w)


def _t_body(s_ref, w_ref, o_ref):
    s = s_ref[0] + s_ref[1]
    o_ref[...] = jnp.dot(s, w_ref[...], preferred_element_type=_f32)


def _tc_t(s_part, w):
    bn = 1000
    grid = NN // bn
    return pl.pallas_call(
        _t_body,
        grid=(grid,),
        in_specs=[
            pl.BlockSpec((NC, bn, H), lambda i: (0, i, 0)),
            pl.BlockSpec((H, H), lambda i: (0, 0)),
        ],
        out_specs=pl.BlockSpec((bn, H), lambda i: (i, 0)),
        out_shape=jax.ShapeDtypeStruct((NN, H), _f32),
    )(s_part, w)


def _h_body(h_ref, s_ref, a_ref, w1_ref, w2_ref, b_ref, o_ref):
    agg = s_ref[0] + s_ref[1] + a_ref[0] + a_ref[1]
    o = (jnp.dot(h_ref[...], w1_ref[...], preferred_element_type=_f32)
         + jnp.dot(agg, w2_ref[...], preferred_element_type=_f32)
         + b_ref[...])
    o_ref[...] = jnp.maximum(o, 0.0)


def _tc_h(h, s_part, a_part, w1, w2, b):
    bn = 1000
    grid = NN // bn
    return pl.pallas_call(
        _h_body,
        grid=(grid,),
        in_specs=[
            pl.BlockSpec((bn, H), lambda i: (i, 0)),
            pl.BlockSpec((NC, bn, H), lambda i: (0, i, 0)),
            pl.BlockSpec((NC, bn, H), lambda i: (0, i, 0)),
            pl.BlockSpec((H, H), lambda i: (0, 0)),
            pl.BlockSpec((H, H), lambda i: (0, 0)),
            pl.BlockSpec((1, H), lambda i: (0, 0)),
        ],
        out_specs=pl.BlockSpec((bn, H), lambda i: (i, 0)),
        out_shape=jax.ShapeDtypeStruct((NN, H), _f32),
    )(h, s_part, a_part, w1, w2, b)


def _fc_body(h_ref, w_ref, b_ref, o_ref):
    o = jnp.dot(h_ref[...], w_ref[...], preferred_element_type=_f32) + b_ref[...]
    o_ref[...] = jnp.maximum(o, 0.0)


def _tc_fc(h, w, b):
    bn = 1000
    grid = NN // bn
    return pl.pallas_call(
        _fc_body,
        grid=(grid,),
        in_specs=[
            pl.BlockSpec((bn, H), lambda i: (i, 0)),
            pl.BlockSpec((H, H), lambda i: (0, 0)),
            pl.BlockSpec((1, H), lambda i: (0, 0)),
        ],
        out_specs=pl.BlockSpec((bn, H), lambda i: (i, 0)),
        out_shape=jax.ShapeDtypeStruct((NN, H), _f32),
    )(h, w, b)


def kernel(x, edge_index, W_ne0, b_ne0, W_ee0, b_ee0, W_en0, b_en0,
           W_ne1, b_ne1, W_ee1, b_ee1, W_en1, b_en1,
           W_ne2, b_ne2, W_ee2, b_ee2, W_en2, b_en2, W_fc, b_fc):
    src = edge_index[0]
    dst = edge_index[1]
    inv, _ = _sc_degree(src, dst)

    layers = [
        (W_ne0, b_ne0, W_ee0, b_ee0, W_en0, b_en0),
        (W_ne1, b_ne1, W_ee1, b_ee1, W_en1, b_en1),
        (W_ne2, b_ne2, W_ee2, b_ee2, W_en2, b_en2),
    ]
    h = x
    Rm = None
    for l, (W_ne, b_ne, W_ee, b_ee, W_en, b_en) in enumerate(layers):
        Wa, Wb = W_ne[:H], W_ne[H:2 * H]
        ca = b_ne + (W_ne[2 * H] if l == 0 else 0.0)
        P, Q = _tc_pq(h, Wa, Wb, ca.reshape(1, H))
        if l == 0:
            he, s_part = _sc_edge_up0(P, Q, src, dst)
        else:
            he, s_part = _sc_edge_up1(P, Q, Rm, src, dst)
        T = _tc_t(s_part, W_ee)
        U2 = _tc_mm2(he, W_ee)
        if l < 2:
            rp, a_part = _sc_edge_ref_rp(T, U2, inv, b_ee, src, dst)
            Wc_next = layers[l + 1][0][2 * H:]
            Rm = _tc_heb_mm(rp, he, Wc_next)
        else:
            (a_part,) = _sc_edge_ref_last(T, U2, inv, b_ee, src, dst)
        h = _tc_h(h, s_part, a_part, W_en[:H], W_en[H:], b_en.reshape(1, H))
    return _tc_fc(h, W_fc, b_fc.reshape(1, H))
